# baseline (speedup 1.0000x reference)
import functools

import jax
import jax.numpy as jnp
from jax import lax
from jax.experimental import pallas as pl
from jax.experimental.pallas import tpu as pltpu
from jax.experimental.pallas import tpu_sc as plsc

CUTOFF = 12.0
CUTON = 0.8 * CUTOFF
KE = 14.399645

N_NODES = 50000
N_EDGES = 1600000
NW = 32
E_PER_W = N_EDGES // NW
B = 2000
NCH = E_PER_W // B
LANES = 16

_F16_SCALE = 5.192296858534828e33
_HI = jnp.int32(-65536)


def _f16_lo(w):
    b = w & 0xFFFF
    return plsc.bitcast(((b & 0x8000) << 16) | ((b & 0x7FFF) << 13),
                        jnp.float32) * _F16_SCALE


def _body(uv_hbm, w1_hbm, w2_hbm, ta_hbm, tb_hbm, out_hbm,
          uv_v, w1_v, w2_v, o_v, ta_v, tb_v, sem_in, sem_out):
    wid = lax.axis_index("s") * 2 + lax.axis_index("c")

    pltpu.sync_copy(ta_hbm, ta_v)
    pltpu.sync_copy(tb_hbm, tb_v)

    c_shift_a = 2.0 / CUTOFF
    c_shift_b = 1.0 / (CUTOFF * CUTOFF)
    inv_w = 1.0 / (CUTOFF - CUTON)

    def in_pairs(base, bb):
        return [(uv_hbm.at[pl.ds(base, B)], uv_v.at[pl.ds(bb, B)]),
                (w1_hbm.at[pl.ds(base, B)], w1_v.at[pl.ds(bb, B)]),
                (w2_hbm.at[pl.ds(base, B)], w2_v.at[pl.ds(bb, B)])]

    def issue_in(ci, bb):
        base = wid * E_PER_W + ci * B
        for src, dst in in_pairs(base, bb):
            pltpu.async_copy(src, dst, sem_in)

    issue_in(0, 0)

    def g_body(g, carry):
        bb = (g & 1) * B
        base = wid * E_PER_W + g * B

        @pl.when(g + 1 < NCH)
        def _prefetch():
            issue_in(g + 1, B - bb)

        for src, dst in in_pairs(base, bb):
            pltpu.make_async_copy(src, dst, sem_in).wait()

        @pl.when(g >= 2)
        def _guard():
            pltpu.make_async_copy(o_v.at[pl.ds(bb, B)],
                                  out_hbm.at[pl.ds(base, B)], sem_out).wait()

        @plsc.parallel_loop(0, B, step=LANES, unroll=4)
        def step(s0):
            s = bb + s0
            w = uv_v[pl.ds(s, LANES)]
            iu = w & 0xFFFF
            iv = lax.shift_right_logical(w, 16)
            wa_u = plsc.load_gather(ta_v, [iu])
            wa_v = plsc.load_gather(ta_v, [iv])
            wb_u = plsc.load_gather(tb_v, [iu])
            qu = _f16_lo(wa_u)
            qv = _f16_lo(wa_v)
            dz = plsc.bitcast(wa_u & _HI, jnp.float32)
            dx = plsc.bitcast(wb_u << 16, jnp.float32)
            dy = plsc.bitcast(wb_u & _HI, jnp.float32)

            w1 = w1_v[pl.ds(s, LANES)]
            w2 = w2_v[pl.ds(s, LANES)]
            vx = plsc.bitcast(w1 << 16, jnp.float32)
            vy = plsc.bitcast(w1 & _HI, jnp.float32)
            vz = plsc.bitcast(w2 & _HI, jnp.float32)
            d = _f16_lo(w2)

            chi = 1.0 / d
            chi_shift = c_shift_a - d * c_shift_b
            e = qu * qv * (chi - chi_shift)
            chi2 = chi * chi
            chi2_shift = chi_shift * chi_shift
            dot = (vx * dx + vy * dy + vz * dz) * chi
            e = e + qv * dot * (chi2 - chi2_shift)
            x = (d - CUTON) * inv_w
            x = jnp.minimum(jnp.maximum(x, 0.0), 1.0)
            sw = 1.0 + x * x * x * (-10.0 + x * (15.0 - 6.0 * x))
            o_v[pl.ds(s, LANES)] = (KE * e) * sw

        pltpu.async_copy(o_v.at[pl.ds(bb, B)],
                         out_hbm.at[pl.ds(base, B)], sem_out)
        return carry

    lax.fori_loop(0, NCH, g_body, 0)

    pltpu.make_async_copy(o_v.at[pl.ds(0, B)],
                          out_hbm.at[pl.ds(0, B)], sem_out).wait()
    pltpu.make_async_copy(o_v.at[pl.ds(0, B)],
                          out_hbm.at[pl.ds(0, B)], sem_out).wait()


def _b16(x, dtype):
    return lax.bitcast_convert_type(
        x.astype(dtype), jnp.uint16).astype(jnp.int32)


def kernel(mlmm_distances, mlmm_vectors, mlmm_atomic_charges, atomic_dipoles,
           mlmm_idxu, mlmm_idxv):
    iu = mlmm_idxu.astype(jnp.int32)
    iv = mlmm_idxv.astype(jnp.int32)
    uv = iu | (iv << 16)

    vec_t = mlmm_vectors.T
    w1 = _b16(vec_t[0], jnp.bfloat16) | (_b16(vec_t[1], jnp.bfloat16) << 16)
    w2 = _b16(mlmm_distances, jnp.float16) | (_b16(vec_t[2], jnp.bfloat16) << 16)

    q16 = _b16(mlmm_atomic_charges, jnp.float16)
    dip_t = atomic_dipoles.T
    d16 = [_b16(dip_t[i], jnp.bfloat16) for i in range(3)]
    word_a = q16 | (d16[2] << 16)
    word_b = d16[0] | (d16[1] << 16)

    mesh = plsc.VectorSubcoreMesh(core_axis_name="c", subcore_axis_name="s")
    run = functools.partial(
        pl.kernel,
        out_type=jax.ShapeDtypeStruct((N_EDGES,), jnp.float32),
        mesh=mesh,
        compiler_params=pltpu.CompilerParams(
            needs_layout_passes=False, use_tc_tiling_on_sc=False),
        scratch_types=[
            pltpu.VMEM((2 * B,), jnp.int32),
            pltpu.VMEM((2 * B,), jnp.int32),
            pltpu.VMEM((2 * B,), jnp.int32),
            pltpu.VMEM((2 * B,), jnp.float32),
            pltpu.VMEM((N_NODES,), jnp.int32),
            pltpu.VMEM((N_NODES,), jnp.int32),
            pltpu.SemaphoreType.DMA,
            pltpu.SemaphoreType.DMA,
        ],
    )(_body)
    return run(uv, w1, w2, word_a, word_b)


# speedup vs baseline: 1.0598x; 1.0598x over previous
import functools

import jax
import jax.numpy as jnp
from jax import lax
from jax.experimental import pallas as pl
from jax.experimental.pallas import tpu as pltpu
from jax.experimental.pallas import tpu_sc as plsc

CUTOFF = 12.0
CUTON = 0.8 * CUTOFF
KE = 14.399645

N_NODES = 50000
N_EDGES = 1600000
NW = 32
E_PER_W = N_EDGES // NW
B = 2000
NCH = E_PER_W // B
LANES = 16

_HI = jnp.int32(-65536)


def _body(uv_hbm, d_hbm, vx_hbm, vy_hbm, vz_hbm, ta_hbm, tb_hbm, out_hbm,
          uv_v, d_v, vx_v, vy_v, vz_v, o_v, ta_v, tb_v, sem_in, sem_out):
    wid = lax.axis_index("s") * 2 + lax.axis_index("c")

    pltpu.sync_copy(ta_hbm, ta_v)
    pltpu.sync_copy(tb_hbm, tb_v)

    c_shift_a = 2.0 / CUTOFF
    c_shift_b = 1.0 / (CUTOFF * CUTOFF)
    inv_w = 1.0 / (CUTOFF - CUTON)

    def in_pairs(base, bb):
        return [(uv_hbm.at[pl.ds(base, B)], uv_v.at[pl.ds(bb, B)]),
                (d_hbm.at[pl.ds(base, B)], d_v.at[pl.ds(bb, B)]),
                (vx_hbm.at[pl.ds(base, B)], vx_v.at[pl.ds(bb, B)]),
                (vy_hbm.at[pl.ds(base, B)], vy_v.at[pl.ds(bb, B)]),
                (vz_hbm.at[pl.ds(base, B)], vz_v.at[pl.ds(bb, B)])]

    def issue_in(ci, bb):
        base = wid * E_PER_W + ci * B
        for src, dst in in_pairs(base, bb):
            pltpu.async_copy(src, dst, sem_in)

    issue_in(0, 0)

    def g_body(g, carry):
        bb = (g & 1) * B
        base = wid * E_PER_W + g * B

        @pl.when(g + 1 < NCH)
        def _prefetch():
            issue_in(g + 1, B - bb)

        for src, dst in in_pairs(base, bb):
            pltpu.make_async_copy(src, dst, sem_in).wait()

        @pl.when(g >= 2)
        def _guard():
            pltpu.make_async_copy(o_v.at[pl.ds(bb, B)],
                                  out_hbm.at[pl.ds(base, B)], sem_out).wait()

        @plsc.parallel_loop(0, B, step=LANES, unroll=8)
        def step(s0):
            s = bb + s0
            w = uv_v[pl.ds(s, LANES)]
            iu = w & 0xFFFF
            iv = lax.shift_right_logical(w, 16)
            wa_u = plsc.load_gather(ta_v, [iu])
            wa_v = plsc.load_gather(ta_v, [iv])
            wb_u = plsc.load_gather(tb_v, [iu])
            qu = plsc.bitcast(wa_u << 16, jnp.float32)
            qv = plsc.bitcast(wa_v << 16, jnp.float32)
            dz = plsc.bitcast(wa_u & _HI, jnp.float32)
            dx = plsc.bitcast(wb_u << 16, jnp.float32)
            dy = plsc.bitcast(wb_u & _HI, jnp.float32)

            d = d_v[pl.ds(s, LANES)]
            vx = vx_v[pl.ds(s, LANES)]
            vy = vy_v[pl.ds(s, LANES)]
            vz = vz_v[pl.ds(s, LANES)]

            chi = 1.0 / d
            chi_shift = c_shift_a - d * c_shift_b
            e = qu * qv * (chi - chi_shift)
            chi2 = chi * chi
            chi2_shift = chi_shift * chi_shift
            dot = (vx * dx + vy * dy + vz * dz) * chi
            e = e + qv * dot * (chi2 - chi2_shift)
            x = (d - CUTON) * inv_w
            x = jnp.minimum(jnp.maximum(x, 0.0), 1.0)
            sw = 1.0 + x * x * x * (-10.0 + x * (15.0 - 6.0 * x))
            o_v[pl.ds(s, LANES)] = (KE * e) * sw

        pltpu.async_copy(o_v.at[pl.ds(bb, B)],
                         out_hbm.at[pl.ds(base, B)], sem_out)
        return carry

    lax.fori_loop(0, NCH, g_body, 0)

    pltpu.make_async_copy(o_v.at[pl.ds(0, B)],
                          out_hbm.at[pl.ds(0, B)], sem_out).wait()
    pltpu.make_async_copy(o_v.at[pl.ds(0, B)],
                          out_hbm.at[pl.ds(0, B)], sem_out).wait()


def _b16(x, dtype):
    return lax.bitcast_convert_type(
        x.astype(dtype), jnp.uint16).astype(jnp.int32)


def kernel(mlmm_distances, mlmm_vectors, mlmm_atomic_charges, atomic_dipoles,
           mlmm_idxu, mlmm_idxv):
    iu = mlmm_idxu.astype(jnp.int32)
    iv = mlmm_idxv.astype(jnp.int32)
    uv = iu | (iv << 16)

    vec_t = mlmm_vectors.T
    vx, vy, vz = vec_t[0], vec_t[1], vec_t[2]

    q16 = _b16(mlmm_atomic_charges, jnp.bfloat16)
    dip_t = atomic_dipoles.T
    d16 = [_b16(dip_t[i], jnp.bfloat16) for i in range(3)]
    word_a = q16 | (d16[2] << 16)
    word_b = d16[0] | (d16[1] << 16)

    mesh = plsc.VectorSubcoreMesh(core_axis_name="c", subcore_axis_name="s")
    run = functools.partial(
        pl.kernel,
        out_type=jax.ShapeDtypeStruct((N_EDGES,), jnp.float32),
        mesh=mesh,
        compiler_params=pltpu.CompilerParams(
            needs_layout_passes=False, use_tc_tiling_on_sc=False),
        scratch_types=[
            pltpu.VMEM((2 * B,), jnp.int32),
            pltpu.VMEM((2 * B,), jnp.float32),
            pltpu.VMEM((2 * B,), jnp.float32),
            pltpu.VMEM((2 * B,), jnp.float32),
            pltpu.VMEM((2 * B,), jnp.float32),
            pltpu.VMEM((2 * B,), jnp.float32),
            pltpu.VMEM((N_NODES,), jnp.int32),
            pltpu.VMEM((N_NODES,), jnp.int32),
            pltpu.SemaphoreType.DMA,
            pltpu.SemaphoreType.DMA,
        ],
    )(_body)
    return run(uv, mlmm_distances, vx, vy, vz, word_a, word_b)


# speedup vs baseline: 1.0821x; 1.0211x over previous
import functools

import jax
import jax.numpy as jnp
from jax import lax
from jax.experimental import pallas as pl
from jax.experimental.pallas import tpu as pltpu
from jax.experimental.pallas import tpu_sc as plsc

CUTOFF = 12.0
CUTON = 0.8 * CUTOFF
KE = 14.399645

N_NODES = 50000
N_EDGES = 1600000
NW = 32
E_PER_W = N_EDGES // NW
B = 2000
NCH = E_PER_W // B
LANES = 16

_HI = jnp.int32(-65536)
_RND = jnp.int32(0x8000)

ROWS = 12500
BR = 1250


def _sc_body(uv_hbm, ta_hbm, tb_hbm, qq_hbm, g1_hbm, g2_hbm,
             uv_v, qq_v, g1_v, g2_v, ta_v, tb_v, sem_in, sem_out):
    wid = lax.axis_index("s") * 2 + lax.axis_index("c")

    pltpu.sync_copy(ta_hbm, ta_v)
    pltpu.sync_copy(tb_hbm, tb_v)

    def out_pairs(base, bb):
        return [(qq_v.at[pl.ds(bb, B)], qq_hbm.at[pl.ds(base, B)]),
                (g1_v.at[pl.ds(bb, B)], g1_hbm.at[pl.ds(base, B)]),
                (g2_v.at[pl.ds(bb, B)], g2_hbm.at[pl.ds(base, B)])]

    def issue_in(ci, bb):
        base = wid * E_PER_W + ci * B
        pltpu.async_copy(uv_hbm.at[pl.ds(base, B)],
                         uv_v.at[pl.ds(bb, B)], sem_in)

    issue_in(0, 0)

    def g_body(g, carry):
        bb = (g & 1) * B
        base = wid * E_PER_W + g * B

        @pl.when(g + 1 < NCH)
        def _prefetch():
            issue_in(g + 1, B - bb)

        pltpu.make_async_copy(uv_hbm.at[pl.ds(base, B)],
                              uv_v.at[pl.ds(bb, B)], sem_in).wait()

        @pl.when(g >= 2)
        def _guard():
            for src, dst in out_pairs(base, bb):
                pltpu.make_async_copy(src, dst, sem_out).wait()

        @plsc.parallel_loop(0, B, step=LANES, unroll=8)
        def step(s0):
            s = bb + s0
            w = uv_v[pl.ds(s, LANES)]
            iu = w & 0xFFFF
            iv = lax.shift_right_logical(w, 16)
            wa_u = plsc.load_gather(ta_v, [iu])
            wa_v = plsc.load_gather(ta_v, [iv])
            wb_u = plsc.load_gather(tb_v, [iu])
            qu = plsc.bitcast(wa_u << 16, jnp.float32)
            qv = plsc.bitcast(wa_v << 16, jnp.float32)
            dz = plsc.bitcast(wa_u & _HI, jnp.float32)
            dx = plsc.bitcast(wb_u << 16, jnp.float32)
            dy = plsc.bitcast(wb_u & _HI, jnp.float32)

            qq_v[pl.ds(s, LANES)] = qu * qv
            gx = plsc.bitcast(qv * dx, jnp.int32)
            gy = plsc.bitcast(qv * dy, jnp.int32)
            gz = plsc.bitcast(qv * dz, jnp.int32)
            g1_v[pl.ds(s, LANES)] = (
                lax.shift_right_logical(gx + _RND, 16) | ((gy + _RND) & _HI))
            g2_v[pl.ds(s, LANES)] = lax.shift_right_logical(gz + _RND, 16)

        for src, dst in out_pairs(base, bb):
            pltpu.async_copy(src, dst, sem_out)
        return carry

    lax.fori_loop(0, NCH, g_body, 0)

    for _ in range(2):
        for ref, hbm in [(qq_v, qq_hbm), (g1_v, g1_hbm), (g2_v, g2_hbm)]:
            pltpu.make_async_copy(ref.at[pl.ds(0, B)],
                                  hbm.at[pl.ds(0, B)], sem_out).wait()


def _tc_body(d_ref, vx_ref, vy_ref, vz_ref, qq_ref, g1_ref, g2_ref, o_ref):
    c_shift_a = 2.0 / CUTOFF
    c_shift_b = 1.0 / (CUTOFF * CUTOFF)
    inv_w = 1.0 / (CUTOFF - CUTON)

    d = d_ref[...]
    g1 = g1_ref[...]
    g2 = g2_ref[...]
    bc = lambda x: lax.bitcast_convert_type(x, jnp.float32)
    gx = bc(g1 << 16)
    gy = bc(g1 & (-65536))
    gz = bc(g2 << 16)

    chi = 1.0 / d
    chi_shift = c_shift_a - d * c_shift_b
    e = qq_ref[...] * (chi - chi_shift)
    chi2 = chi * chi
    chi2_shift = chi_shift * chi_shift
    dot = (vx_ref[...] * gx + vy_ref[...] * gy + vz_ref[...] * gz) * chi
    e = e + dot * (chi2 - chi2_shift)
    x = (d - CUTON) * inv_w
    x = jnp.minimum(jnp.maximum(x, 0.0), 1.0)
    sw = 1.0 + x * x * x * (-10.0 + x * (15.0 - 6.0 * x))
    o_ref[...] = (KE * e) * sw


def _b16(x, dtype):
    return lax.bitcast_convert_type(
        x.astype(dtype), jnp.uint16).astype(jnp.int32)


def kernel(mlmm_distances, mlmm_vectors, mlmm_atomic_charges, atomic_dipoles,
           mlmm_idxu, mlmm_idxv):
    iu = mlmm_idxu.astype(jnp.int32)
    iv = mlmm_idxv.astype(jnp.int32)
    uv = iu | (iv << 16)

    q16 = _b16(mlmm_atomic_charges, jnp.bfloat16)
    dip_t = atomic_dipoles.T
    d16 = [_b16(dip_t[i], jnp.bfloat16) for i in range(3)]
    word_a = q16 | (d16[2] << 16)
    word_b = d16[0] | (d16[1] << 16)

    vec_t = mlmm_vectors.T
    vx, vy, vz = vec_t[0], vec_t[1], vec_t[2]

    mesh = plsc.VectorSubcoreMesh(core_axis_name="c", subcore_axis_name="s")
    sc_run = functools.partial(
        pl.kernel,
        out_type=(jax.ShapeDtypeStruct((N_EDGES,), jnp.float32),
                  jax.ShapeDtypeStruct((N_EDGES,), jnp.int32),
                  jax.ShapeDtypeStruct((N_EDGES,), jnp.int32)),
        mesh=mesh,
        compiler_params=pltpu.CompilerParams(
            needs_layout_passes=False, use_tc_tiling_on_sc=False),
        scratch_types=[
            pltpu.VMEM((2 * B,), jnp.int32),
            pltpu.VMEM((2 * B,), jnp.float32),
            pltpu.VMEM((2 * B,), jnp.int32),
            pltpu.VMEM((2 * B,), jnp.int32),
            pltpu.VMEM((N_NODES,), jnp.int32),
            pltpu.VMEM((N_NODES,), jnp.int32),
            pltpu.SemaphoreType.DMA,
            pltpu.SemaphoreType.DMA,
        ],
    )(_sc_body)
    qq, g1, g2 = sc_run(uv, word_a, word_b)

    return pl.pallas_call(
        _tc_body,
        out_shape=jax.ShapeDtypeStruct((N_EDGES,), jnp.float32),
    )(mlmm_distances, vx, vy, vz, qq, g1, g2)


# speedup vs baseline: 1.0822x; 1.0001x over previous
import functools

import jax
import jax.numpy as jnp
from jax import lax
from jax.experimental import pallas as pl
from jax.experimental.pallas import tpu as pltpu
from jax.experimental.pallas import tpu_sc as plsc

CUTOFF = 12.0
CUTON = 0.8 * CUTOFF
KE = 14.399645

N_NODES = 50000
N_EDGES = 1600000
NW = 32
E_PER_W = N_EDGES // NW
B = 2000
NCH = E_PER_W // B
LANES = 16

_HI = jnp.int32(-65536)
_RND = jnp.int32(0x8000)

ROWS = 12500
BR = 1250


def _sc_body(uv_hbm, ta_hbm, tb_hbm, qq_hbm, g1_hbm, g2_hbm,
             uv_v, qq_v, g1_v, g2_v, ta_v, tb_v, sem_in, sem_out):
    wid = lax.axis_index("s") * 2 + lax.axis_index("c")

    pltpu.sync_copy(ta_hbm, ta_v)
    pltpu.sync_copy(tb_hbm, tb_v)

    def out_pairs(base, bb):
        return [(qq_v.at[pl.ds(bb, B)], qq_hbm.at[pl.ds(base, B)]),
                (g1_v.at[pl.ds(bb, B)], g1_hbm.at[pl.ds(base, B)]),
                (g2_v.at[pl.ds(bb, B)], g2_hbm.at[pl.ds(base, B)])]

    def issue_in(ci, bb):
        base = wid * E_PER_W + ci * B
        pltpu.async_copy(uv_hbm.at[pl.ds(base, B)],
                         uv_v.at[pl.ds(bb, B)], sem_in)

    issue_in(0, 0)

    def g_body(g, carry):
        bb = (g & 1) * B
        base = wid * E_PER_W + g * B

        @pl.when(g + 1 < NCH)
        def _prefetch():
            issue_in(g + 1, B - bb)

        pltpu.make_async_copy(uv_hbm.at[pl.ds(base, B)],
                              uv_v.at[pl.ds(bb, B)], sem_in).wait()

        @pl.when(g >= 2)
        def _guard():
            for src, dst in out_pairs(base, bb):
                pltpu.make_async_copy(src, dst, sem_out).wait()

        @plsc.parallel_loop(0, B, step=LANES, unroll=8)
        def step(s0):
            s = bb + s0
            w = uv_v[pl.ds(s, LANES)]
            iu = w & 0xFFFF
            iv = lax.shift_right_logical(w, 16)
            wa_u = plsc.load_gather(ta_v, [iu])
            wa_v = plsc.load_gather(ta_v, [iv])
            wb_u = plsc.load_gather(tb_v, [iu])
            qu = plsc.bitcast(wa_u << 16, jnp.float32)
            qv = plsc.bitcast(wa_v << 16, jnp.float32)
            dz = plsc.bitcast(wa_u & _HI, jnp.float32)
            dx = plsc.bitcast(wb_u << 16, jnp.float32)
            dy = plsc.bitcast(wb_u & _HI, jnp.float32)

            qq_v[pl.ds(s, LANES)] = qu * qv
            gx = plsc.bitcast(qv * dx, jnp.int32)
            gy = plsc.bitcast(qv * dy, jnp.int32)
            gz = plsc.bitcast(qv * dz, jnp.int32)
            g1_v[pl.ds(s, LANES)] = (
                lax.shift_right_logical(gx + _RND, 16) | ((gy + _RND) & _HI))
            g2_v[pl.ds(s, LANES)] = lax.shift_right_logical(gz + _RND, 16)

        for src, dst in out_pairs(base, bb):
            pltpu.async_copy(src, dst, sem_out)
        return carry

    lax.fori_loop(0, NCH, g_body, 0)

    for _ in range(2):
        for ref, hbm in [(qq_v, qq_hbm), (g1_v, g1_hbm), (g2_v, g2_hbm)]:
            pltpu.make_async_copy(ref.at[pl.ds(0, B)],
                                  hbm.at[pl.ds(0, B)], sem_out).wait()


def _tc_body(d_ref, vx_ref, vy_ref, vz_ref, qq_ref, g1_ref, g2_ref, o_ref):
    c_shift_a = 2.0 / CUTOFF
    c_shift_b = 1.0 / (CUTOFF * CUTOFF)
    inv_w = 1.0 / (CUTOFF - CUTON)

    d = d_ref[...]
    g1 = g1_ref[...]
    g2 = g2_ref[...]
    bc = lambda x: lax.bitcast_convert_type(x, jnp.float32)
    gx = bc(g1 << 16)
    gy = bc(g1 & (-65536))
    gz = bc(g2 << 16)

    chi = 1.0 / d
    chi_shift = c_shift_a - d * c_shift_b
    e = qq_ref[...] * (chi - chi_shift)
    chi2 = chi * chi
    chi2_shift = chi_shift * chi_shift
    dot = (vx_ref[...] * gx + vy_ref[...] * gy + vz_ref[...] * gz) * chi
    e = e + dot * (chi2 - chi2_shift)
    x = (d - CUTON) * inv_w
    x = jnp.minimum(jnp.maximum(x, 0.0), 1.0)
    sw = 1.0 + x * x * x * (-10.0 + x * (15.0 - 6.0 * x))
    o_ref[...] = (KE * e) * sw


def _b16(x, dtype):
    return lax.bitcast_convert_type(
        x.astype(dtype), jnp.uint16).astype(jnp.int32)


def kernel(mlmm_distances, mlmm_vectors, mlmm_atomic_charges, atomic_dipoles,
           mlmm_idxu, mlmm_idxv):
    iu = mlmm_idxu.astype(jnp.int32)
    iv = mlmm_idxv.astype(jnp.int32)
    uv = iu | (iv << 16)

    q16 = _b16(mlmm_atomic_charges, jnp.bfloat16)
    dip_t = atomic_dipoles.T
    d16 = [_b16(dip_t[i], jnp.bfloat16) for i in range(3)]
    word_a = q16 | (d16[2] << 16)
    word_b = d16[0] | (d16[1] << 16)

    mesh = plsc.VectorSubcoreMesh(core_axis_name="c", subcore_axis_name="s")
    sc_run = functools.partial(
        pl.kernel,
        out_type=(jax.ShapeDtypeStruct((N_EDGES,), jnp.float32),
                  jax.ShapeDtypeStruct((N_EDGES,), jnp.int32),
                  jax.ShapeDtypeStruct((N_EDGES,), jnp.int32)),
        mesh=mesh,
        compiler_params=pltpu.CompilerParams(
            needs_layout_passes=False, use_tc_tiling_on_sc=False),
        scratch_types=[
            pltpu.VMEM((2 * B,), jnp.int32),
            pltpu.VMEM((2 * B,), jnp.float32),
            pltpu.VMEM((2 * B,), jnp.int32),
            pltpu.VMEM((2 * B,), jnp.int32),
            pltpu.VMEM((N_NODES,), jnp.int32),
            pltpu.VMEM((N_NODES,), jnp.int32),
            pltpu.SemaphoreType.DMA,
            pltpu.SemaphoreType.DMA,
        ],
    )(_sc_body)
    qq, g1, g2 = sc_run(uv, word_a, word_b)

    vec_t = mlmm_vectors.T
    vx, vy, vz = vec_t[0], vec_t[1], vec_t[2]

    return pl.pallas_call(
        _tc_body,
        out_shape=jax.ShapeDtypeStruct((N_EDGES,), jnp.float32),
    )(mlmm_distances, vx, vy, vz, qq, g1, g2)
